# baseline (speedup 1.0000x reference)
import jax
import jax.numpy as jnp
from jax import lax
from jax.experimental import pallas as pl
from jax.experimental.pallas import tpu as pltpu

_B, _L, _D = 4, 2048, 4096
_E, _TOPK = 64, 8
_N = _B * _L
_TILE = 512
_GRID = _N // _TILE


def _router_body(x_ref, w_ref, wts_ref, idx_ref, aux_ref, psum_ref, cnt_ref):
    i = pl.program_id(0)

    @pl.when(i == 0)
    def _init():
        psum_ref[...] = jnp.zeros_like(psum_ref)
        cnt_ref[...] = jnp.zeros_like(cnt_ref)

    x = x_ref[...]
    w = w_ref[...]
    logits = lax.dot_general(
        w, x, (((1,), (1,)), ((), ())),
        preferred_element_type=jnp.float32) * 0.5
    m = jnp.max(logits, axis=0, keepdims=True)
    e = jnp.exp(logits - m)
    z = jnp.sum(e, axis=0, keepdims=True)
    probs = e / z
    psum_ref[...] += jnp.sum(probs, axis=1, keepdims=True)

    iota_f = lax.broadcasted_iota(jnp.int32, (_E, _TILE), 0).astype(jnp.float32)
    iota_k = lax.broadcasted_iota(jnp.int32, (_TOPK, _TILE), 0)
    p = probs
    wts = jnp.zeros((_TOPK, _TILE), jnp.float32)
    idxf = jnp.zeros((_TOPK, _TILE), jnp.float32)
    for k in range(_TOPK):
        mk = jnp.max(p, axis=0, keepdims=True)
        t = jnp.where(p == mk, iota_f, float(_E))
        ikf = jnp.min(t, axis=0, keepdims=True)
        wts = jnp.where(iota_k == k, mk, wts)
        idxf = jnp.where(iota_k == k, ikf, idxf)
        p = jnp.where(t == ikf, -1.0, p)
    cnt_ref[...] += jnp.sum((p < 0.0).astype(jnp.float32), axis=1, keepdims=True)
    wsum = jnp.sum(wts, axis=0, keepdims=True) + 1e-8
    wts_ref[...] = wts / wsum
    idx_ref[...] = idxf.astype(jnp.int32)

    @pl.when(i == _GRID - 1)
    def _fin():
        f = cnt_ref[...] / (_N * _TOPK)
        pmean = psum_ref[...] / _N
        aux_ref[...] = _E * jnp.sum(f * pmean, axis=0, keepdims=True)


def kernel(x, W):
    xf = x.reshape(_N, _D)
    wts_t, idx_t, aux = pl.pallas_call(
        _router_body,
        grid=(_GRID,),
        in_specs=[
            pl.BlockSpec((_TILE, _D), lambda i: (i, 0)),
            pl.BlockSpec((_E, _D), lambda i: (0, 0)),
        ],
        out_specs=[
            pl.BlockSpec((_TOPK, _TILE), lambda i: (0, i)),
            pl.BlockSpec((_TOPK, _TILE), lambda i: (0, i)),
            pl.BlockSpec((1, 1), lambda i: (0, 0)),
        ],
        out_shape=[
            jax.ShapeDtypeStruct((_TOPK, _N), jnp.float32),
            jax.ShapeDtypeStruct((_TOPK, _N), jnp.int32),
            jax.ShapeDtypeStruct((1, 1), jnp.float32),
        ],
        scratch_shapes=[
            pltpu.VMEM((_E, 1), jnp.float32),
            pltpu.VMEM((_E, 1), jnp.float32),
        ],
    )(xf, W)
    wts = wts_t.T.reshape(_B, _L, _TOPK)
    idx = idx_t.T.reshape(_B, _L, _TOPK)
    return (wts, idx, aux[0, 0])


# speedup vs baseline: 1.0516x; 1.0516x over previous
import jax
import jax.numpy as jnp
from jax import lax
from jax.experimental import pallas as pl
from jax.experimental.pallas import tpu as pltpu

_B, _L, _D = 4, 2048, 4096
_E, _TOPK = 64, 8
_N = _B * _L
_TILE = 1024
_GRID = _N // _TILE


def _router_body(x_ref, w_ref, wts_ref, idx_ref, aux_ref, psum_ref, cnt_ref):
    i = pl.program_id(0)

    @pl.when(i == 0)
    def _init():
        psum_ref[...] = jnp.zeros_like(psum_ref)
        cnt_ref[...] = jnp.zeros_like(cnt_ref)

    x = x_ref[...]
    w = w_ref[...]
    logits = lax.dot_general(
        w, x, (((1,), (1,)), ((), ())),
        preferred_element_type=jnp.float32) * 0.5
    m = jnp.max(logits, axis=0, keepdims=True)
    e = jnp.exp(logits - m)
    z = jnp.sum(e, axis=0, keepdims=True)
    probs = e / z
    psum_ref[...] += jnp.sum(probs, axis=1, keepdims=True)

    iota_f = lax.broadcasted_iota(jnp.int32, (_E, _TILE), 0).astype(jnp.float32)
    iota_k = lax.broadcasted_iota(jnp.int32, (_TOPK, _TILE), 0)
    p = probs
    wts = jnp.zeros((_TOPK, _TILE), jnp.float32)
    idxf = jnp.zeros((_TOPK, _TILE), jnp.float32)
    for k in range(_TOPK):
        mk = jnp.max(p, axis=0, keepdims=True)
        t = jnp.where(p == mk, iota_f, float(_E))
        ikf = jnp.min(t, axis=0, keepdims=True)
        wts = jnp.where(iota_k == k, mk, wts)
        idxf = jnp.where(iota_k == k, ikf, idxf)
        p = jnp.where(t == ikf, -1.0, p)
    cnt_ref[...] += jnp.sum((p < 0.0).astype(jnp.float32), axis=1, keepdims=True)
    wsum = jnp.sum(wts, axis=0, keepdims=True) + 1e-8
    wts_ref[...] = wts / wsum
    idx_ref[...] = idxf.astype(jnp.int32)

    @pl.when(i == _GRID - 1)
    def _fin():
        f = cnt_ref[...] / (_N * _TOPK)
        pmean = psum_ref[...] / _N
        aux_ref[...] = _E * jnp.sum(f * pmean, axis=0, keepdims=True)


def kernel(x, W):
    xf = x.reshape(_N, _D)
    wts_t, idx_t, aux = pl.pallas_call(
        _router_body,
        grid=(_GRID,),
        in_specs=[
            pl.BlockSpec((_TILE, _D), lambda i: (i, 0)),
            pl.BlockSpec((_E, _D), lambda i: (0, 0)),
        ],
        out_specs=[
            pl.BlockSpec((_TOPK, _TILE), lambda i: (0, i)),
            pl.BlockSpec((_TOPK, _TILE), lambda i: (0, i)),
            pl.BlockSpec((1, 1), lambda i: (0, 0)),
        ],
        out_shape=[
            jax.ShapeDtypeStruct((_TOPK, _N), jnp.float32),
            jax.ShapeDtypeStruct((_TOPK, _N), jnp.int32),
            jax.ShapeDtypeStruct((1, 1), jnp.float32),
        ],
        scratch_shapes=[
            pltpu.VMEM((_E, 1), jnp.float32),
            pltpu.VMEM((_E, 1), jnp.float32),
        ],
    )(xf, W)
    wts = wts_t.T.reshape(_B, _L, _TOPK)
    idx = idx_t.T.reshape(_B, _L, _TOPK)
    return (wts, idx, aux[0, 0])
